# baseline (speedup 1.0000x reference)
import functools

import jax
import jax.numpy as jnp
from jax import lax
from jax.experimental import pallas as pl
from jax.experimental.pallas import tpu as pltpu
from jax.experimental.pallas import tpu_sc as plsc

N = 20
D = 64
B = 16384
OUTW = 3 * D
NC = 2
NS = 16
NW = NC * NS
BPW = B // NW
CB = 32
NCHUNK = BPW // CB
IPC = CB * N
GSZ = 128
NG = IPC // GSZ
URO = BPW // GSZ
NVD = D // 16


def _sc_body(mem_idx_hbm, user_hbm, user_table, item_table, wb_hbm, out_hbm,
             idx_v, uidx_v, urows_v, items_v, outb_v, wb_v, sem):
    wid = lax.axis_index("s") * NC + lax.axis_index("c")
    base = wid * BPW

    pltpu.sync_copy(mem_idx_hbm.at[pl.ds(wid * (BPW * N // GSZ), BPW * N // GSZ)],
                    idx_v)
    pltpu.sync_copy(user_hbm.at[pl.ds(wid * URO, URO)], uidx_v)
    pltpu.sync_copy(wb_hbm, wb_v)

    ucps = [pltpu.async_copy(user_table.at[uidx_v.at[r]],
                             urows_v.at[pl.ds(r * GSZ, GSZ)], sem)
            for r in range(URO)]
    for c in ucps:
        c.wait()

    wv = [wb_v[n, :] for n in range(N)]
    bias = wb_v[N, :]

    def chunk(j, carry):
        cps = [pltpu.async_copy(item_table.at[idx_v.at[j * NG + g]],
                                items_v.at[pl.ds(g * GSZ, GSZ)], sem)
               for g in range(NG)]
        for c in cps:
            c.wait()

        def bbody(b, c2):
            row0 = b * N
            ub = j * CB + b
            for d in range(NVD):
                u = urows_v[ub, pl.ds(d * 16, 16)]
                acc = bias
                for n in range(N):
                    acc = acc + wv[n] * items_v[row0 + n, pl.ds(d * 16, 16)]
                outb_v[b, pl.ds(d * 16, 16)] = u
                outb_v[b, pl.ds(D + d * 16, 16)] = u * acc
                outb_v[b, pl.ds(2 * D + d * 16, 16)] = acc
            return c2

        lax.fori_loop(0, CB, bbody, 0, unroll=False)
        pltpu.sync_copy(outb_v, out_hbm.at[pl.ds(base + j * CB, CB)])
        return carry

    lax.fori_loop(0, NCHUNK, chunk, 0, unroll=False)


@jax.jit
def _run(user_r, mem_r, user_table, item_table, wb):
    mesh = plsc.VectorSubcoreMesh(core_axis_name="c", subcore_axis_name="s",
                                  num_cores=NC, num_subcores=NS)
    fn = pl.kernel(
        _sc_body,
        out_type=jax.ShapeDtypeStruct((B, OUTW), jnp.float32),
        mesh=mesh,
        compiler_params=pltpu.CompilerParams(use_tc_tiling_on_sc=False),
        scratch_types=[
            pltpu.VMEM((BPW * N // GSZ, GSZ), jnp.int32),
            pltpu.VMEM((URO, GSZ), jnp.int32),
            pltpu.VMEM((BPW, D), jnp.float32),
            pltpu.VMEM((IPC, D), jnp.float32),
            pltpu.VMEM((CB, OUTW), jnp.float32),
            pltpu.VMEM((N + 1, 16), jnp.float32),
            pltpu.SemaphoreType.DMA,
        ],
    )
    return fn(mem_r, user_r, user_table, item_table, wb)


def kernel(user, memory, user_table, item_table, conv_w, conv_b):
    w = conv_w.reshape(N)
    wb = jnp.broadcast_to(jnp.concatenate([w, conv_b]).reshape(N + 1, 1),
                          (N + 1, 16)).astype(jnp.float32)
    mem_r = memory.astype(jnp.int32).reshape(B * N // GSZ, GSZ)
    user_r = user.astype(jnp.int32).reshape(B // GSZ, GSZ)
    return _run(user_r, mem_r, user_table, item_table, wb)
